# baseline (speedup 1.0000x reference)
import jax
import jax.numpy as jnp
from jax.experimental import pallas as pl
from jax.experimental.pallas import tpu as pltpu

_OUT_DIM = 200
_BLOCK_B = 4096
_SLOTS = 4


def _mlp_tile_kernel(x_ref, w1_ref, b1_ref, w2_ref, b2_ref, out_ref,
                     q_buf, sems):
    i = pl.program_id(0)
    n = pl.num_programs(0)
    slot = jax.lax.rem(i, _SLOTS)

    def _store(step, wait):
        cp = pltpu.make_async_copy(
            q_buf.at[jax.lax.rem(step, _SLOTS)],
            out_ref.at[pl.ds(step * _BLOCK_B, _BLOCK_B), :],
            sems.at[jax.lax.rem(step, _SLOTS)],
        )
        cp.wait() if wait else cp.start()

    @pl.when(i >= _SLOTS)
    def _():
        _store(i - _SLOTS, wait=True)

    h = jnp.dot(x_ref[...], w1_ref[...], preferred_element_type=jnp.float32)
    h = jnp.maximum(h + b1_ref[...], 0.0)
    q = jnp.dot(h, w2_ref[...], preferred_element_type=jnp.float32)
    q_buf[slot] = q + b2_ref[...]

    _store(i, wait=False)

    @pl.when(i == n - 1)
    def _():
        for back in range(_SLOTS - 1, -1, -1):
            _store(n - 1 - back, wait=True)


@jax.jit
def kernel(x, w1_t, b1, w2_t, b2):
    B, S = x.shape
    Hp = w1_t.shape[1]
    O = _OUT_DIM
    w2_s = w2_t[:, :O]
    b2_s = b2[:, :O]

    grid = (B // _BLOCK_B,)
    return pl.pallas_call(
        _mlp_tile_kernel,
        out_shape=jax.ShapeDtypeStruct((B, O), jnp.float32),
        grid=grid,
        in_specs=[
            pl.BlockSpec((_BLOCK_B, S), lambda i: (i, 0)),
            pl.BlockSpec((S, Hp), lambda i: (0, 0)),
            pl.BlockSpec((1, Hp), lambda i: (0, 0)),
            pl.BlockSpec((Hp, O), lambda i: (0, 0)),
            pl.BlockSpec((1, O), lambda i: (0, 0)),
        ],
        out_specs=pl.BlockSpec(memory_space=pltpu.MemorySpace.HBM),
        scratch_shapes=[
            pltpu.VMEM((_SLOTS, _BLOCK_B, O), jnp.float32),
            pltpu.SemaphoreType.DMA((_SLOTS,)),
        ],
        compiler_params=pltpu.CompilerParams(
            dimension_semantics=("arbitrary",),
        ),
        cost_estimate=pl.CostEstimate(
            flops=2 * B * (S * Hp + Hp * O),
            transcendentals=0,
            bytes_accessed=4 * (B * S + S * Hp + Hp + Hp * O + O + B * O),
        ),
    )(x, w1_t, b1, w2_s, b2_s)


# speedup vs baseline: 1.1310x; 1.1310x over previous
import jax
import jax.numpy as jnp
from jax.experimental import pallas as pl
from jax.experimental.pallas import tpu as pltpu

_BLOCK_B = 8192


def _mlp_tile_kernel(x_ref, w1_ref, b1_ref, w2_ref, b2_ref, out_ref):
    h = jnp.dot(x_ref[...], w1_ref[...], preferred_element_type=jnp.float32)
    h = jnp.maximum(h + b1_ref[...], 0.0)
    q = jnp.dot(h, w2_ref[...], preferred_element_type=jnp.float32)
    out_ref[...] = q + b2_ref[...]


@jax.jit
def kernel(x, w1_t, b1, w2_t, b2):
    B, S = x.shape
    Hp = w1_t.shape[1]
    Op = w2_t.shape[1]
    grid = (B // _BLOCK_B,)
    y = pl.pallas_call(
        _mlp_tile_kernel,
        out_shape=jax.ShapeDtypeStruct((B, Op), jnp.float32),
        grid=grid,
        in_specs=[
            pl.BlockSpec((_BLOCK_B, S), lambda i: (i, 0)),
            pl.BlockSpec((S, Hp), lambda i: (0, 0)),
            pl.BlockSpec((1, Hp), lambda i: (0, 0)),
            pl.BlockSpec((Hp, Op), lambda i: (0, 0)),
            pl.BlockSpec((1, Op), lambda i: (0, 0)),
        ],
        out_specs=pl.BlockSpec((_BLOCK_B, Op), lambda i: (i, 0)),
        compiler_params=pltpu.CompilerParams(dimension_semantics=("parallel",)),
    )(x, w1_t, b1, w2_t, b2)
    return y[:, :200]
